# Initial kernel scaffold; baseline (speedup 1.0000x reference)
import jax
import jax.numpy as jnp
from jax.experimental import pallas as pl


def kernel(inputs):
    raise NotImplementedError("write your pallas kernel here")


# speedup vs baseline: 360.7253x; 360.7253x over previous
import functools

import jax
import jax.numpy as jnp
from jax import lax
from jax.experimental import pallas as pl
from jax.experimental.pallas import tpu as pltpu
from jax.experimental.pallas import tpu_sc as plsc

ROWS = 64
COLS = 32768
BINS = 1024
L = 16
NC = 2
NS = 16
NW = NC * NS
CHUNK = 8


def _body(in_hbm, out_hbm, vals_v, out_v, tbl_v, sem):
    wid = lax.axis_index("s") * NC + lax.axis_index("c")
    rows_per_w = ROWS // NW
    ones = jnp.ones((L,), jnp.float32)
    zeros = jnp.zeros((L,), jnp.float32)

    def do_row(r, carry):
        row = wid * rows_per_w + r
        pltpu.sync_copy(in_hbm.at[row], vals_v)

        def zbody(i, c):
            tbl_v[pl.ds(i * L, L)] = zeros
            return c

        lax.fori_loop(0, BINS // L, zbody, 0)

        def hbody(i, c):
            base = i * (L * CHUNK)
            for k in range(CHUNK):
                v = vals_v[pl.ds(base + k * L, L)]
                plsc.addupdate_scatter(tbl_v, [v], ones)
            return c

        lax.fori_loop(0, COLS // (L * CHUNK), hbody, 0)

        def ibody(i, c):
            cnt = tbl_v[pl.ds(i * L, L)]
            tbl_v[pl.ds(i * L, L)] = 1.0 / cnt
            return c

        lax.fori_loop(0, BINS // L, ibody, 0)

        def gbody(i, c):
            base = i * (L * CHUNK)
            for k in range(CHUNK):
                v = vals_v[pl.ds(base + k * L, L)]
                out_v[pl.ds(base + k * L, L)] = plsc.load_gather(tbl_v, [v])
            return c

        lax.fori_loop(0, COLS // (L * CHUNK), gbody, 0)

        pltpu.sync_copy(out_v, out_hbm.at[row])
        return carry

    lax.fori_loop(0, rows_per_w, do_row, 0)


@jax.jit
def kernel(inputs):
    k = pl.kernel(
        _body,
        out_type=jax.ShapeDtypeStruct((ROWS, COLS), jnp.float32),
        mesh=plsc.VectorSubcoreMesh(core_axis_name="c", subcore_axis_name="s"),
        scratch_types=[
            pltpu.VMEM((COLS,), jnp.int32),
            pltpu.VMEM((COLS,), jnp.float32),
            pltpu.VMEM((BINS,), jnp.float32),
            pltpu.SemaphoreType.DMA,
        ],
        compiler_params=pltpu.CompilerParams(needs_layout_passes=False),
    )
    return k(inputs.astype(jnp.int32))


# speedup vs baseline: 361.8288x; 1.0031x over previous
import functools

import jax
import jax.numpy as jnp
from jax import lax
from jax.experimental import pallas as pl
from jax.experimental.pallas import tpu as pltpu
from jax.experimental.pallas import tpu_sc as plsc

ROWS = 64
COLS = 32768
BINS = 1024
L = 16
NC = 2
NS = 16
NW = NC * NS
CHUNK = 16


def _body(in_hbm, out_hbm, vals_v, out_v, tbl_v, sem):
    wid = lax.axis_index("s") * NC + lax.axis_index("c")
    rows_per_w = ROWS // NW
    ones = jnp.ones((L,), jnp.float32)
    zeros = jnp.zeros((L,), jnp.float32)

    def do_row(r, carry):
        row = wid * rows_per_w + r
        pltpu.sync_copy(in_hbm.at[row], vals_v)

        def zbody(i, c):
            tbl_v[pl.ds(i * L, L)] = zeros
            return c

        lax.fori_loop(0, BINS // L, zbody, 0)

        def hbody(i, c):
            base = i * (L * CHUNK)
            for k in range(CHUNK):
                v = vals_v[pl.ds(base + k * L, L)]
                plsc.addupdate_scatter(tbl_v, [v], ones)
            return c

        lax.fori_loop(0, COLS // (L * CHUNK), hbody, 0)

        def ibody(i, c):
            cnt = tbl_v[pl.ds(i * L, L)]
            tbl_v[pl.ds(i * L, L)] = 1.0 / cnt
            return c

        lax.fori_loop(0, BINS // L, ibody, 0)

        def gbody(i, c):
            base = i * (L * CHUNK)
            for k in range(CHUNK):
                v = vals_v[pl.ds(base + k * L, L)]
                out_v[pl.ds(base + k * L, L)] = plsc.load_gather(tbl_v, [v])
            return c

        lax.fori_loop(0, COLS // (L * CHUNK), gbody, 0)

        pltpu.sync_copy(out_v, out_hbm.at[row])
        return carry

    lax.fori_loop(0, rows_per_w, do_row, 0)


@jax.jit
def kernel(inputs):
    k = pl.kernel(
        _body,
        out_type=jax.ShapeDtypeStruct((ROWS, COLS), jnp.float32),
        mesh=plsc.VectorSubcoreMesh(core_axis_name="c", subcore_axis_name="s"),
        scratch_types=[
            pltpu.VMEM((COLS,), jnp.int32),
            pltpu.VMEM((COLS,), jnp.float32),
            pltpu.VMEM((BINS,), jnp.float32),
            pltpu.SemaphoreType.DMA,
        ],
        compiler_params=pltpu.CompilerParams(needs_layout_passes=False),
    )
    return k(inputs.astype(jnp.int32))


# speedup vs baseline: 622.7460x; 1.7211x over previous
import functools

import jax
import jax.numpy as jnp
from jax import lax
from jax.experimental import pallas as pl
from jax.experimental.pallas import tpu as pltpu
from jax.experimental.pallas import tpu_sc as plsc

ROWS = 64
COLS = 32768
BINS = 1024
L = 16
NC = 2
NS = 16
NW = NC * NS
CHUNK = 16


def _body(in_hbm, out_hbm, vals_v, out_v, tbl_v, sem):
    wid = lax.axis_index("s") * NC + lax.axis_index("c")
    rows_per_w = ROWS // NW
    ones = jnp.ones((L,), jnp.float32)
    zeros = jnp.zeros((L,), jnp.float32)

    for r in range(rows_per_w):
        row = wid * rows_per_w + r
        pltpu.sync_copy(in_hbm.at[row], vals_v)

        @plsc.parallel_loop(0, BINS, step=L, unroll=4)
        def zloop(i):
            tbl_v[pl.ds(i, L)] = zeros

        @plsc.parallel_loop(0, COLS, step=L, unroll=CHUNK)
        def hloop(i):
            v = vals_v[pl.ds(i, L)]
            plsc.addupdate_scatter(tbl_v, [v], ones)

        @plsc.parallel_loop(0, BINS, step=L, unroll=4)
        def iloop(i):
            tbl_v[pl.ds(i, L)] = 1.0 / tbl_v[pl.ds(i, L)]

        @plsc.parallel_loop(0, COLS, step=L, unroll=CHUNK)
        def gloop(i):
            v = vals_v[pl.ds(i, L)]
            out_v[pl.ds(i, L)] = plsc.load_gather(tbl_v, [v])

        pltpu.sync_copy(out_v, out_hbm.at[row])


@jax.jit
def kernel(inputs):
    k = pl.kernel(
        _body,
        out_type=jax.ShapeDtypeStruct((ROWS, COLS), jnp.float32),
        mesh=plsc.VectorSubcoreMesh(core_axis_name="c", subcore_axis_name="s"),
        scratch_types=[
            pltpu.VMEM((COLS,), jnp.int32),
            pltpu.VMEM((COLS,), jnp.float32),
            pltpu.VMEM((BINS,), jnp.float32),
            pltpu.SemaphoreType.DMA,
        ],
        compiler_params=pltpu.CompilerParams(needs_layout_passes=False),
    )
    return k(inputs.astype(jnp.int32))


# speedup vs baseline: 662.0182x; 1.0631x over previous
import functools

import jax
import jax.numpy as jnp
from jax import lax
from jax.experimental import pallas as pl
from jax.experimental.pallas import tpu as pltpu
from jax.experimental.pallas import tpu_sc as plsc

ROWS = 64
COLS = 32768
BINS = 1024
L = 16
NC = 2
NS = 16
NW = NC * NS
CHUNK = 16


def _zero_tbl(tbl_v):
    zeros = jnp.zeros((L,), jnp.float32)

    @plsc.parallel_loop(0, BINS, step=L, unroll=4)
    def zloop(i):
        tbl_v[pl.ds(i, L)] = zeros


def _hist(vals_v, tbl_v):
    ones = jnp.ones((L,), jnp.float32)

    @plsc.parallel_loop(0, COLS, step=L, unroll=CHUNK)
    def hloop(i):
        v = vals_v[pl.ds(i, L)]
        plsc.addupdate_scatter(tbl_v, [v], ones)


def _invert(tbl_v):
    @plsc.parallel_loop(0, BINS, step=L, unroll=4)
    def iloop(i):
        tbl_v[pl.ds(i, L)] = 1.0 / tbl_v[pl.ds(i, L)]


def _gather(vals_v, tbl_v, out_v):
    @plsc.parallel_loop(0, COLS, step=L, unroll=CHUNK)
    def gloop(i):
        v = vals_v[pl.ds(i, L)]
        out_v[pl.ds(i, L)] = plsc.load_gather(tbl_v, [v])


def _body(in_hbm, out_hbm, vals0_v, vals1_v, out_v, tbl_v, sem0, sem1, semo):
    wid = lax.axis_index("s") * NC + lax.axis_index("c")
    row0 = wid * 2
    row1 = row0 + 1

    cp0 = pltpu.async_copy(in_hbm.at[row0], vals0_v, sem0)
    cp1 = pltpu.async_copy(in_hbm.at[row1], vals1_v, sem1)

    cp0.wait()
    _zero_tbl(tbl_v)
    _hist(vals0_v, tbl_v)
    _invert(tbl_v)
    _gather(vals0_v, tbl_v, out_v)
    ocp = pltpu.async_copy(out_v, out_hbm.at[row0], semo)

    cp1.wait()
    _zero_tbl(tbl_v)
    _hist(vals1_v, tbl_v)
    _invert(tbl_v)
    ocp.wait()
    _gather(vals1_v, tbl_v, out_v)
    pltpu.sync_copy(out_v, out_hbm.at[row1])


@jax.jit
def kernel(inputs):
    k = pl.kernel(
        _body,
        out_type=jax.ShapeDtypeStruct((ROWS, COLS), jnp.float32),
        mesh=plsc.VectorSubcoreMesh(core_axis_name="c", subcore_axis_name="s"),
        scratch_types=[
            pltpu.VMEM((COLS,), jnp.int32),
            pltpu.VMEM((COLS,), jnp.int32),
            pltpu.VMEM((COLS,), jnp.float32),
            pltpu.VMEM((BINS,), jnp.float32),
            pltpu.SemaphoreType.DMA,
            pltpu.SemaphoreType.DMA,
            pltpu.SemaphoreType.DMA,
        ],
        compiler_params=pltpu.CompilerParams(needs_layout_passes=False),
    )
    return k(inputs.astype(jnp.int32))
